# baseline (speedup 1.0000x reference)
import functools

import jax
import jax.numpy as jnp
from jax import lax
from jax.experimental import pallas as pl
from jax.experimental.pallas import tpu as pltpu
from jax.experimental.pallas import tpu_sc as plsc

NC = 2
NS = 16
NW = NC * NS
L = 16

BATCH = 4096
HIST = 200
EMBED = 64
VOCAB = 1000000

VBLK = 256
NBLK = VOCAB // VBLK
VTAIL = VOCAB - NBLK * VBLK
OUTER_K = (NBLK + 2 * NW - 1) // (2 * NW)

C2 = BATCH // NW
NSLOT = 3

_params = pltpu.CompilerParams(use_tc_tiling_on_sc=True,
                               needs_layout_passes=False)


@functools.lru_cache(maxsize=None)
def _make_transpose_kernel():
    mesh = plsc.VectorSubcoreMesh(core_axis_name="c", subcore_axis_name="s")

    @functools.partial(
        pl.kernel,
        mesh=mesh,
        out_type=jax.ShapeDtypeStruct((VOCAB // 2, 2 * EMBED), jnp.float32),
        scratch_types=[
            pltpu.VMEM((2, EMBED, VBLK), jnp.float32),
            pltpu.VMEM((2, VBLK // 2, 2 * EMBED), jnp.float32),
            pltpu.SemaphoreType.DMA((2,)),
            pltpu.SemaphoreType.DMA((2,)),
        ],
        compiler_params=_params,
    )
    def transpose_kernel(wt, wtail, table, sbuf, tbuf, sem_r, sem_w):
        wid = lax.axis_index("s") * NC + lax.axis_index("c")
        iota = lax.iota(jnp.int32, L)
        perms = [(iota + k) & (L - 1) for k in range(L)]
        lanebase = (iota & 1) * EMBED

        def start_read(b, slot):
            pltpu.async_copy(wt.at[pl.ds(0, EMBED), pl.ds(b * VBLK, VBLK)],
                             sbuf.at[slot], sem_r.at[slot])

        def block_of(k, slot):
            return (k * 2 + slot) * NW + wid

        def transpose_block(slot):
            def vg_body(vg, carry):
                col_vec = vg * L + iota
                p_vec = col_vec >> 1
                for eg in range(EMBED // L):
                    for k0 in range(0, L, 8):
                        rows = [eg * L + perms[k0 + i] for i in range(8)]
                        vals = [plsc.load_gather(sbuf.at[slot],
                                                 [rv, col_vec])
                                for rv in rows]
                        for rv, v in zip(rows, vals):
                            plsc.store_scatter(tbuf.at[slot],
                                               [p_vec, lanebase + rv], v)
                return carry
            lax.fori_loop(0, VBLK // L, vg_body, 0)

        start_read(block_of(0, 0), 0)
        start_read(block_of(0, 1), 1)

        def outer(k, carry):
            for slot in range(2):
                b = block_of(k, slot)

                @pl.when(b < NBLK)
                def _():
                    pltpu.make_async_copy(
                        wt.at[pl.ds(0, EMBED), pl.ds(b * VBLK, VBLK)],
                        sbuf.at[slot], sem_r.at[slot]).wait()

                    @pl.when(k > 0)
                    def _():
                        pltpu.make_async_copy(
                            tbuf.at[slot],
                            table.at[pl.ds(b * (VBLK // 2), VBLK // 2),
                                     pl.ds(0, 2 * EMBED)],
                            sem_w.at[slot]).wait()

                    transpose_block(slot)
                    pltpu.async_copy(
                        tbuf.at[slot],
                        table.at[pl.ds(b * (VBLK // 2), VBLK // 2),
                                 pl.ds(0, 2 * EMBED)],
                        sem_w.at[slot])
                    nb = block_of(k + 1, slot)

                    @pl.when(nb < NBLK)
                    def _():
                        start_read(nb, slot)
            return carry

        lax.fori_loop(0, OUTER_K, outer, 0)

        for slot in range(2):
            last_k = (NBLK - 1 - (wid + slot * NW)) // (2 * NW)
            b = block_of(last_k, slot)

            @pl.when(b < NBLK)
            def _():
                pltpu.make_async_copy(
                    tbuf.at[slot],
                    table.at[pl.ds(b * (VBLK // 2), VBLK // 2),
                             pl.ds(0, 2 * EMBED)],
                    sem_w.at[slot]).wait()

        @pl.when(wid == 0)
        def _():
            pltpu.async_copy(wtail,
                             sbuf.at[0, pl.ds(0, EMBED), pl.ds(0, 2 * EMBED)],
                             sem_r.at[0])
            pltpu.make_async_copy(
                wtail, sbuf.at[0, pl.ds(0, EMBED), pl.ds(0, 2 * EMBED)],
                sem_r.at[0]).wait()
            transpose_block(0)
            pltpu.async_copy(
                tbuf.at[0, pl.ds(0, VTAIL // 2), pl.ds(0, 2 * EMBED)],
                table.at[pl.ds(NBLK * (VBLK // 2), VTAIL // 2),
                         pl.ds(0, 2 * EMBED)],
                sem_w.at[0])
            pltpu.make_async_copy(
                tbuf.at[0, pl.ds(0, VTAIL // 2), pl.ds(0, 2 * EMBED)],
                table.at[pl.ds(NBLK * (VBLK // 2), VTAIL // 2),
                         pl.ds(0, 2 * EMBED)],
                sem_w.at[0]).wait()

    return transpose_kernel


@functools.lru_cache(maxsize=None)
def _make_gather_kernel():
    mesh = plsc.VectorSubcoreMesh(core_axis_name="c", subcore_axis_name="s")

    @functools.partial(
        pl.kernel,
        mesh=mesh,
        out_type=jax.ShapeDtypeStruct((HIST, EMBED, BATCH), jnp.float32),
        scratch_types=[
            pltpu.VMEM((HIST, C2), jnp.int32),
            pltpu.VMEM((NSLOT, 1, C2), jnp.int32),
            pltpu.VMEM((NSLOT, C2, 2 * EMBED), jnp.float32),
            pltpu.VMEM((NSLOT, EMBED, C2), jnp.float32),
            pltpu.SemaphoreType.DMA((NSLOT,)),
            pltpu.SemaphoreType.DMA((NSLOT,)),
            pltpu.SemaphoreType.DMA,
        ],
        compiler_params=_params,
    )
    def gather_kernel(idxT, table, out, idx_v, pidx_v, rows_v, t_v, sem_g,
                      sem_w, sem_i):
        wid = lax.axis_index("s") * NC + lax.axis_index("c")
        b0 = wid * C2
        iota = lax.iota(jnp.int32, L)
        perms = [(iota + k) & (L - 1) for k in range(L)]

        pltpu.async_copy(idxT.at[pl.ds(0, HIST), pl.ds(b0, C2)], idx_v,
                         sem_i).wait()

        def start_gather(h, slot):
            for cg in range(C2 // L):
                idx = idx_v[h, pl.ds(cg * L, L)]
                pidx_v[slot, 0, pl.ds(cg * L, L)] = idx >> 1
            pltpu.async_copy(table.at[pidx_v.at[slot, 0]], rows_v.at[slot],
                             sem_g.at[slot])

        def transpose_unit(h, gslot, tslot):
            def cg_body(cg, carry):
                c_vec = cg * L + iota
                idx = idx_v[h, pl.ds(cg * L, L)]
                colbase = (idx & 1) * EMBED
                for eg in range(EMBED // L):
                    for k0 in range(0, L, 4):
                        rows = [eg * L + perms[k0 + i] for i in range(4)]
                        vals = [plsc.load_gather(rows_v.at[gslot],
                                                 [c_vec, colbase + rv])
                                for rv in rows]
                        for rv, v in zip(rows, vals):
                            plsc.store_scatter(t_v.at[tslot], [rv, c_vec],
                                               v)
                return carry
            lax.fori_loop(0, C2 // L, cg_body, 0)

        for s in range(NSLOT):
            start_gather(s, s)

        def outer(g, carry):
            for i in range(NSLOT):
                h = g * NSLOT + i

                @pl.when(h < HIST)
                def _():
                    pltpu.make_async_copy(table.at[pidx_v.at[i, 0]],
                                          rows_v.at[i],
                                          sem_g.at[i]).wait()

                    @pl.when(g > 0)
                    def _():
                        pltpu.make_async_copy(
                            t_v.at[i],
                            out.at[h, pl.ds(0, EMBED), pl.ds(b0, C2)],
                            sem_w.at[i]).wait()

                    transpose_unit(h, i, i)
                    pltpu.async_copy(
                        t_v.at[i],
                        out.at[h, pl.ds(0, EMBED), pl.ds(b0, C2)],
                        sem_w.at[i])

                    @pl.when(h + NSLOT < HIST)
                    def _():
                        start_gather(h + NSLOT, i)
            return carry

        lax.fori_loop(0, (HIST + NSLOT - 1) // NSLOT, outer, 0)

        for i in range(NSLOT):
            h_last = ((HIST - 1 - i) // NSLOT) * NSLOT + i

            @pl.when(h_last < HIST)
            def _():
                pltpu.make_async_copy(
                    t_v.at[i],
                    out.at[h_last, pl.ds(0, EMBED), pl.ds(b0, C2)],
                    sem_w.at[i]).wait()

    return gather_kernel


def kernel(input, weight):
    wt = weight.T
    wtail = jnp.pad(wt[:, NBLK * VBLK:], ((0, 0), (0, 2 * EMBED - VTAIL)))
    table = _make_transpose_kernel()(wt, wtail)
    idxT = input.T
    out = _make_gather_kernel()(idxT, table)
    return out.transpose(2, 0, 1)


# speedup vs baseline: 1.0585x; 1.0585x over previous
import functools

import jax
import jax.numpy as jnp
from jax import lax
from jax.experimental import pallas as pl
from jax.experimental.pallas import tpu as pltpu
from jax.experimental.pallas import tpu_sc as plsc

NC = 2
NS = 16
NW = NC * NS
L = 16

BATCH = 4096
HIST = 200
EMBED = 64
VOCAB = 1000000

VBLK = 256
NBLK = VOCAB // VBLK
VTAIL = VOCAB - NBLK * VBLK
OUTER_K = (NBLK + 2 * NW - 1) // (2 * NW)

C2 = BATCH // NW
NSLOT = 4

_params = pltpu.CompilerParams(use_tc_tiling_on_sc=True,
                               needs_layout_passes=False)


@functools.lru_cache(maxsize=None)
def _make_transpose_kernel():
    mesh = plsc.VectorSubcoreMesh(core_axis_name="c", subcore_axis_name="s")

    @functools.partial(
        pl.kernel,
        mesh=mesh,
        out_type=jax.ShapeDtypeStruct((VOCAB // 2, 2 * EMBED), jnp.float32),
        scratch_types=[
            pltpu.VMEM((2, EMBED, VBLK), jnp.float32),
            pltpu.VMEM((2, VBLK // 2, 2 * EMBED), jnp.float32),
            pltpu.SemaphoreType.DMA((2,)),
            pltpu.SemaphoreType.DMA((2,)),
        ],
        compiler_params=_params,
    )
    def transpose_kernel(wt, wtail, table, sbuf, tbuf, sem_r, sem_w):
        wid = lax.axis_index("s") * NC + lax.axis_index("c")
        iota = lax.iota(jnp.int32, L)
        perms = [(iota + k) & (L - 1) for k in range(L)]
        lanebase = (iota & 1) * EMBED

        def start_read(b, slot):
            pltpu.async_copy(wt.at[pl.ds(0, EMBED), pl.ds(b * VBLK, VBLK)],
                             sbuf.at[slot], sem_r.at[slot])

        def block_of(k, slot):
            return (k * 2 + slot) * NW + wid

        def transpose_block(slot):
            def vg_body(vg, carry):
                col_vec = vg * L + iota
                p_vec = col_vec >> 1
                for eg in range(EMBED // L):
                    for k0 in range(0, L, 4):
                        rows = [eg * L + perms[k0 + i] for i in range(4)]
                        vals = [plsc.load_gather(sbuf.at[slot],
                                                 [rv, col_vec])
                                for rv in rows]
                        for rv, v in zip(rows, vals):
                            plsc.store_scatter(tbuf.at[slot],
                                               [p_vec, lanebase + rv], v)
                return carry
            lax.fori_loop(0, VBLK // L, vg_body, 0)

        start_read(block_of(0, 0), 0)
        start_read(block_of(0, 1), 1)

        def outer(k, carry):
            for slot in range(2):
                b = block_of(k, slot)

                @pl.when(b < NBLK)
                def _():
                    pltpu.make_async_copy(
                        wt.at[pl.ds(0, EMBED), pl.ds(b * VBLK, VBLK)],
                        sbuf.at[slot], sem_r.at[slot]).wait()

                    @pl.when(k > 0)
                    def _():
                        pltpu.make_async_copy(
                            tbuf.at[slot],
                            table.at[pl.ds(b * (VBLK // 2), VBLK // 2),
                                     pl.ds(0, 2 * EMBED)],
                            sem_w.at[slot]).wait()

                    transpose_block(slot)
                    pltpu.async_copy(
                        tbuf.at[slot],
                        table.at[pl.ds(b * (VBLK // 2), VBLK // 2),
                                 pl.ds(0, 2 * EMBED)],
                        sem_w.at[slot])
                    nb = block_of(k + 1, slot)

                    @pl.when(nb < NBLK)
                    def _():
                        start_read(nb, slot)
            return carry

        lax.fori_loop(0, OUTER_K, outer, 0)

        for slot in range(2):
            last_k = (NBLK - 1 - (wid + slot * NW)) // (2 * NW)
            b = block_of(last_k, slot)

            @pl.when(b < NBLK)
            def _():
                pltpu.make_async_copy(
                    tbuf.at[slot],
                    table.at[pl.ds(b * (VBLK // 2), VBLK // 2),
                             pl.ds(0, 2 * EMBED)],
                    sem_w.at[slot]).wait()

        @pl.when(wid == 0)
        def _():
            pltpu.async_copy(wtail,
                             sbuf.at[0, pl.ds(0, EMBED), pl.ds(0, 2 * EMBED)],
                             sem_r.at[0])
            pltpu.make_async_copy(
                wtail, sbuf.at[0, pl.ds(0, EMBED), pl.ds(0, 2 * EMBED)],
                sem_r.at[0]).wait()
            transpose_block(0)
            pltpu.async_copy(
                tbuf.at[0, pl.ds(0, VTAIL // 2), pl.ds(0, 2 * EMBED)],
                table.at[pl.ds(NBLK * (VBLK // 2), VTAIL // 2),
                         pl.ds(0, 2 * EMBED)],
                sem_w.at[0])
            pltpu.make_async_copy(
                tbuf.at[0, pl.ds(0, VTAIL // 2), pl.ds(0, 2 * EMBED)],
                table.at[pl.ds(NBLK * (VBLK // 2), VTAIL // 2),
                         pl.ds(0, 2 * EMBED)],
                sem_w.at[0]).wait()

    return transpose_kernel


@functools.lru_cache(maxsize=None)
def _make_gather_kernel():
    mesh = plsc.VectorSubcoreMesh(core_axis_name="c", subcore_axis_name="s")

    @functools.partial(
        pl.kernel,
        mesh=mesh,
        out_type=jax.ShapeDtypeStruct((HIST, EMBED, BATCH), jnp.float32),
        scratch_types=[
            pltpu.VMEM((HIST, C2), jnp.int32),
            pltpu.VMEM((NSLOT, 1, C2), jnp.int32),
            pltpu.VMEM((NSLOT, C2, 2 * EMBED), jnp.float32),
            pltpu.VMEM((NSLOT, EMBED, C2), jnp.float32),
            pltpu.SemaphoreType.DMA((NSLOT,)),
            pltpu.SemaphoreType.DMA((NSLOT,)),
            pltpu.SemaphoreType.DMA,
        ],
        compiler_params=_params,
    )
    def gather_kernel(idxT, table, out, idx_v, pidx_v, rows_v, t_v, sem_g,
                      sem_w, sem_i):
        wid = lax.axis_index("s") * NC + lax.axis_index("c")
        b0 = wid * C2
        iota = lax.iota(jnp.int32, L)
        perms = [(iota + k) & (L - 1) for k in range(L)]

        pltpu.async_copy(idxT.at[pl.ds(0, HIST), pl.ds(b0, C2)], idx_v,
                         sem_i).wait()

        def start_gather(h, slot):
            for cg in range(C2 // L):
                idx = idx_v[h, pl.ds(cg * L, L)]
                pidx_v[slot, 0, pl.ds(cg * L, L)] = idx >> 1
            pltpu.async_copy(table.at[pidx_v.at[slot, 0]], rows_v.at[slot],
                             sem_g.at[slot])

        def transpose_unit(h, gslot, tslot):
            def cg_body(cg, carry):
                c_vec = cg * L + iota
                idx = idx_v[h, pl.ds(cg * L, L)]
                colbase = (idx & 1) * EMBED
                for eg in range(EMBED // L):
                    for k0 in range(0, L, 4):
                        rows = [eg * L + perms[k0 + i] for i in range(4)]
                        vals = [plsc.load_gather(rows_v.at[gslot],
                                                 [c_vec, colbase + rv])
                                for rv in rows]
                        for rv, v in zip(rows, vals):
                            plsc.store_scatter(t_v.at[tslot], [rv, c_vec],
                                               v)
                return carry
            lax.fori_loop(0, C2 // L, cg_body, 0)

        for s in range(NSLOT):
            start_gather(s, s)

        def outer(g, carry):
            for i in range(NSLOT):
                h = g * NSLOT + i

                @pl.when(h < HIST)
                def _():
                    pltpu.make_async_copy(table.at[pidx_v.at[i, 0]],
                                          rows_v.at[i],
                                          sem_g.at[i]).wait()

                    @pl.when(g > 0)
                    def _():
                        pltpu.make_async_copy(
                            t_v.at[i],
                            out.at[h, pl.ds(0, EMBED), pl.ds(b0, C2)],
                            sem_w.at[i]).wait()

                    transpose_unit(h, i, i)
                    pltpu.async_copy(
                        t_v.at[i],
                        out.at[h, pl.ds(0, EMBED), pl.ds(b0, C2)],
                        sem_w.at[i])

                    @pl.when(h + NSLOT < HIST)
                    def _():
                        start_gather(h + NSLOT, i)
            return carry

        lax.fori_loop(0, (HIST + NSLOT - 1) // NSLOT, outer, 0)

        for i in range(NSLOT):
            h_last = ((HIST - 1 - i) // NSLOT) * NSLOT + i

            @pl.when(h_last < HIST)
            def _():
                pltpu.make_async_copy(
                    t_v.at[i],
                    out.at[h_last, pl.ds(0, EMBED), pl.ds(b0, C2)],
                    sem_w.at[i]).wait()

    return gather_kernel


def kernel(input, weight):
    wt = weight.T
    wtail = jnp.pad(wt[:, NBLK * VBLK:], ((0, 0), (0, 2 * EMBED - VTAIL)))
    table = _make_transpose_kernel()(wt, wtail)
    idxT = input.T
    out = _make_gather_kernel()(idxT, table)
    return out.transpose(2, 0, 1)


# speedup vs baseline: 1.1061x; 1.0450x over previous
import functools

import jax
import jax.numpy as jnp
from jax import lax
from jax.experimental import pallas as pl
from jax.experimental.pallas import tpu as pltpu
from jax.experimental.pallas import tpu_sc as plsc

NC = 2
NS = 16
NW = NC * NS
L = 16

BATCH = 4096
HIST = 200
EMBED = 64
VOCAB = 1000000

VBLK = 384
NBLK = VOCAB // VBLK
VTAIL = VOCAB - NBLK * VBLK
OUTER_K = (NBLK + 2 * NW - 1) // (2 * NW)

C2 = BATCH // NW
NSLOT = 3

_params = pltpu.CompilerParams(use_tc_tiling_on_sc=True,
                               needs_layout_passes=False)


@functools.lru_cache(maxsize=None)
def _make_transpose_kernel():
    mesh = plsc.VectorSubcoreMesh(core_axis_name="c", subcore_axis_name="s")

    @functools.partial(
        pl.kernel,
        mesh=mesh,
        out_type=jax.ShapeDtypeStruct((VOCAB // 2, 2 * EMBED), jnp.float32),
        scratch_types=[
            pltpu.VMEM((2, EMBED, VBLK), jnp.float32),
            pltpu.VMEM((2, VBLK // 2, 2 * EMBED), jnp.float32),
            pltpu.SemaphoreType.DMA((2,)),
            pltpu.SemaphoreType.DMA((2,)),
        ],
        compiler_params=_params,
    )
    def transpose_kernel(wt, wtail, table, sbuf, tbuf, sem_r, sem_w):
        wid = lax.axis_index("s") * NC + lax.axis_index("c")
        iota = lax.iota(jnp.int32, L)
        perms = [(iota + k) & (L - 1) for k in range(L)]
        lanebase = (iota & 1) * EMBED

        def start_read(b, slot):
            pltpu.async_copy(wt.at[pl.ds(0, EMBED), pl.ds(b * VBLK, VBLK)],
                             sbuf.at[slot], sem_r.at[slot])

        def block_of(k, slot):
            return (k * 2 + slot) * NW + wid

        def transpose_block(slot):
            def vg_body(vg, carry):
                col_vec = vg * L + iota
                p_vec = col_vec >> 1
                for eg in range(EMBED // L):
                    for k0 in range(0, L, 4):
                        rows = [eg * L + perms[k0 + i] for i in range(4)]
                        vals = [plsc.load_gather(sbuf.at[slot],
                                                 [rv, col_vec])
                                for rv in rows]
                        for rv, v in zip(rows, vals):
                            plsc.store_scatter(tbuf.at[slot],
                                               [p_vec, lanebase + rv], v)
                return carry
            lax.fori_loop(0, VBLK // L, vg_body, 0)

        start_read(block_of(0, 0), 0)
        start_read(block_of(0, 1), 1)

        def outer(k, carry):
            for slot in range(2):
                b = block_of(k, slot)

                @pl.when(b < NBLK)
                def _():
                    pltpu.make_async_copy(
                        wt.at[pl.ds(0, EMBED), pl.ds(b * VBLK, VBLK)],
                        sbuf.at[slot], sem_r.at[slot]).wait()

                    @pl.when(k > 0)
                    def _():
                        pltpu.make_async_copy(
                            tbuf.at[slot],
                            table.at[pl.ds(b * (VBLK // 2), VBLK // 2),
                                     pl.ds(0, 2 * EMBED)],
                            sem_w.at[slot]).wait()

                    transpose_block(slot)
                    pltpu.async_copy(
                        tbuf.at[slot],
                        table.at[pl.ds(b * (VBLK // 2), VBLK // 2),
                                 pl.ds(0, 2 * EMBED)],
                        sem_w.at[slot])
                    nb = block_of(k + 1, slot)

                    @pl.when(nb < NBLK)
                    def _():
                        start_read(nb, slot)
            return carry

        lax.fori_loop(0, OUTER_K, outer, 0)

        for slot in range(2):
            last_k = (NBLK - 1 - (wid + slot * NW)) // (2 * NW)
            b = block_of(last_k, slot)

            @pl.when(b < NBLK)
            def _():
                pltpu.make_async_copy(
                    tbuf.at[slot],
                    table.at[pl.ds(b * (VBLK // 2), VBLK // 2),
                             pl.ds(0, 2 * EMBED)],
                    sem_w.at[slot]).wait()

        @pl.when(wid == 0)
        def _():
            pltpu.async_copy(wtail,
                             sbuf.at[0, pl.ds(0, EMBED), pl.ds(0, 2 * EMBED)],
                             sem_r.at[0])
            pltpu.make_async_copy(
                wtail, sbuf.at[0, pl.ds(0, EMBED), pl.ds(0, 2 * EMBED)],
                sem_r.at[0]).wait()
            transpose_block(0)
            pltpu.async_copy(
                tbuf.at[0, pl.ds(0, VTAIL // 2), pl.ds(0, 2 * EMBED)],
                table.at[pl.ds(NBLK * (VBLK // 2), VTAIL // 2),
                         pl.ds(0, 2 * EMBED)],
                sem_w.at[0])
            pltpu.make_async_copy(
                tbuf.at[0, pl.ds(0, VTAIL // 2), pl.ds(0, 2 * EMBED)],
                table.at[pl.ds(NBLK * (VBLK // 2), VTAIL // 2),
                         pl.ds(0, 2 * EMBED)],
                sem_w.at[0]).wait()

    return transpose_kernel


@functools.lru_cache(maxsize=None)
def _make_gather_kernel():
    mesh = plsc.VectorSubcoreMesh(core_axis_name="c", subcore_axis_name="s")

    @functools.partial(
        pl.kernel,
        mesh=mesh,
        out_type=jax.ShapeDtypeStruct((HIST, EMBED, BATCH), jnp.float32),
        scratch_types=[
            pltpu.VMEM((HIST, C2), jnp.int32),
            pltpu.VMEM((NSLOT, 1, C2), jnp.int32),
            pltpu.VMEM((NSLOT, C2, 2 * EMBED), jnp.float32),
            pltpu.VMEM((NSLOT, EMBED, C2), jnp.float32),
            pltpu.SemaphoreType.DMA((NSLOT,)),
            pltpu.SemaphoreType.DMA((NSLOT,)),
            pltpu.SemaphoreType.DMA,
        ],
        compiler_params=_params,
    )
    def gather_kernel(idxT, table, out, idx_v, pidx_v, rows_v, t_v, sem_g,
                      sem_w, sem_i):
        wid = lax.axis_index("s") * NC + lax.axis_index("c")
        b0 = wid * C2
        iota = lax.iota(jnp.int32, L)
        perms = [(iota + k) & (L - 1) for k in range(L)]

        pltpu.async_copy(idxT.at[pl.ds(0, HIST), pl.ds(b0, C2)], idx_v,
                         sem_i).wait()

        def start_gather(h, slot):
            for cg in range(C2 // L):
                idx = idx_v[h, pl.ds(cg * L, L)]
                pidx_v[slot, 0, pl.ds(cg * L, L)] = idx >> 1
            pltpu.async_copy(table.at[pidx_v.at[slot, 0]], rows_v.at[slot],
                             sem_g.at[slot])

        def transpose_unit(h, gslot, tslot):
            def cg_body(cg, carry):
                c_vec = cg * L + iota
                idx = idx_v[h, pl.ds(cg * L, L)]
                colbase = (idx & 1) * EMBED
                for eg in range(EMBED // L):
                    for k0 in range(0, L, 4):
                        rows = [eg * L + perms[k0 + i] for i in range(4)]
                        vals = [plsc.load_gather(rows_v.at[gslot],
                                                 [c_vec, colbase + rv])
                                for rv in rows]
                        for rv, v in zip(rows, vals):
                            plsc.store_scatter(t_v.at[tslot], [rv, c_vec],
                                               v)
                return carry
            lax.fori_loop(0, C2 // L, cg_body, 0)

        for s in range(NSLOT):
            start_gather(s, s)

        def outer(g, carry):
            for i in range(NSLOT):
                h = g * NSLOT + i

                @pl.when(h < HIST)
                def _():
                    pltpu.make_async_copy(table.at[pidx_v.at[i, 0]],
                                          rows_v.at[i],
                                          sem_g.at[i]).wait()

                    @pl.when(g > 0)
                    def _():
                        pltpu.make_async_copy(
                            t_v.at[i],
                            out.at[h, pl.ds(0, EMBED), pl.ds(b0, C2)],
                            sem_w.at[i]).wait()

                    transpose_unit(h, i, i)
                    pltpu.async_copy(
                        t_v.at[i],
                        out.at[h, pl.ds(0, EMBED), pl.ds(b0, C2)],
                        sem_w.at[i])

                    @pl.when(h + NSLOT < HIST)
                    def _():
                        start_gather(h + NSLOT, i)
            return carry

        lax.fori_loop(0, (HIST + NSLOT - 1) // NSLOT, outer, 0)

        for i in range(NSLOT):
            h_last = ((HIST - 1 - i) // NSLOT) * NSLOT + i

            @pl.when(h_last < HIST)
            def _():
                pltpu.make_async_copy(
                    t_v.at[i],
                    out.at[h_last, pl.ds(0, EMBED), pl.ds(b0, C2)],
                    sem_w.at[i]).wait()

    return gather_kernel


def kernel(input, weight):
    wt = weight.T
    wtail = jnp.pad(wt[:, NBLK * VBLK:], ((0, 0), (0, 2 * EMBED - VTAIL)))
    table = _make_transpose_kernel()(wt, wtail)
    idxT = input.T
    out = _make_gather_kernel()(idxT, table)
    return out.transpose(2, 0, 1)


# speedup vs baseline: 1.1398x; 1.0304x over previous
import functools

import jax
import jax.numpy as jnp
from jax import lax
from jax.experimental import pallas as pl
from jax.experimental.pallas import tpu as pltpu
from jax.experimental.pallas import tpu_sc as plsc

NC = 2
NS = 16
NW = NC * NS
L = 16

BATCH = 4096
HIST = 200
EMBED = 64
VOCAB = 1000000

VBLK = 256
NBLK = VOCAB // VBLK
VTAIL = VOCAB - NBLK * VBLK
OUTER_K = (NBLK + 2 * NW - 1) // (2 * NW)

C2 = BATCH // NW
NSLOT = 3

_params = pltpu.CompilerParams(use_tc_tiling_on_sc=True,
                               needs_layout_passes=False)


@functools.lru_cache(maxsize=None)
def _make_transpose_kernel():
    mesh = plsc.VectorSubcoreMesh(core_axis_name="c", subcore_axis_name="s")

    @functools.partial(
        pl.kernel,
        mesh=mesh,
        out_type=jax.ShapeDtypeStruct((VOCAB // 2, 2 * EMBED), jnp.float32),
        scratch_types=[
            pltpu.VMEM((2, EMBED, VBLK), jnp.float32),
            pltpu.VMEM((2, VBLK // 2, 2 * EMBED), jnp.float32),
            pltpu.SemaphoreType.DMA((2,)),
            pltpu.SemaphoreType.DMA((2,)),
        ],
        compiler_params=_params,
    )
    def transpose_kernel(wt, wtail, table, sbuf, tbuf, sem_r, sem_w):
        wid = lax.axis_index("s") * NC + lax.axis_index("c")
        iota = lax.iota(jnp.int32, L)
        perms = [(iota + k) & (L - 1) for k in range(L)]
        lanebase = (iota & 1) * EMBED

        def start_read(b, slot):
            pltpu.async_copy(wt.at[pl.ds(0, EMBED), pl.ds(b * VBLK, VBLK)],
                             sbuf.at[slot], sem_r.at[slot])

        def block_of(k, slot):
            return (k * 2 + slot) * NW + wid

        def transpose_block(slot):
            def vg_body(vg, carry):
                col_vec = vg * L + iota
                p_vec = col_vec >> 1
                for eg in range(EMBED // L):
                    for k0 in range(0, L, 4):
                        rows = [eg * L + perms[k0 + i] for i in range(4)]
                        vals = [plsc.load_gather(sbuf.at[slot],
                                                 [rv, col_vec])
                                for rv in rows]
                        for rv, v in zip(rows, vals):
                            plsc.store_scatter(tbuf.at[slot],
                                               [p_vec, lanebase + rv], v)
                return carry
            lax.fori_loop(0, VBLK // L, vg_body, 0)

        start_read(block_of(0, 0), 0)
        start_read(block_of(0, 1), 1)

        def outer(k, carry):
            for slot in range(2):
                b = block_of(k, slot)

                @pl.when(b < NBLK)
                def _():
                    pltpu.make_async_copy(
                        wt.at[pl.ds(0, EMBED), pl.ds(b * VBLK, VBLK)],
                        sbuf.at[slot], sem_r.at[slot]).wait()

                    @pl.when(k > 0)
                    def _():
                        pltpu.make_async_copy(
                            tbuf.at[slot],
                            table.at[pl.ds(b * (VBLK // 2), VBLK // 2),
                                     pl.ds(0, 2 * EMBED)],
                            sem_w.at[slot]).wait()

                    transpose_block(slot)
                    pltpu.async_copy(
                        tbuf.at[slot],
                        table.at[pl.ds(b * (VBLK // 2), VBLK // 2),
                                 pl.ds(0, 2 * EMBED)],
                        sem_w.at[slot])
                    nb = block_of(k + 1, slot)

                    @pl.when(nb < NBLK)
                    def _():
                        start_read(nb, slot)
            return carry

        lax.fori_loop(0, OUTER_K, outer, 0)

        for slot in range(2):
            last_k = (NBLK - 1 - (wid + slot * NW)) // (2 * NW)
            b = block_of(last_k, slot)

            @pl.when(b < NBLK)
            def _():
                pltpu.make_async_copy(
                    tbuf.at[slot],
                    table.at[pl.ds(b * (VBLK // 2), VBLK // 2),
                             pl.ds(0, 2 * EMBED)],
                    sem_w.at[slot]).wait()

        @pl.when(wid == 0)
        def _():
            pltpu.async_copy(wtail,
                             sbuf.at[0, pl.ds(0, EMBED), pl.ds(0, 2 * EMBED)],
                             sem_r.at[0])
            pltpu.make_async_copy(
                wtail, sbuf.at[0, pl.ds(0, EMBED), pl.ds(0, 2 * EMBED)],
                sem_r.at[0]).wait()
            transpose_block(0)
            pltpu.async_copy(
                tbuf.at[0, pl.ds(0, VTAIL // 2), pl.ds(0, 2 * EMBED)],
                table.at[pl.ds(NBLK * (VBLK // 2), VTAIL // 2),
                         pl.ds(0, 2 * EMBED)],
                sem_w.at[0])
            pltpu.make_async_copy(
                tbuf.at[0, pl.ds(0, VTAIL // 2), pl.ds(0, 2 * EMBED)],
                table.at[pl.ds(NBLK * (VBLK // 2), VTAIL // 2),
                         pl.ds(0, 2 * EMBED)],
                sem_w.at[0]).wait()

    return transpose_kernel


@functools.lru_cache(maxsize=None)
def _make_gather_kernel():
    mesh = plsc.VectorSubcoreMesh(core_axis_name="c", subcore_axis_name="s")

    @functools.partial(
        pl.kernel,
        mesh=mesh,
        out_type=jax.ShapeDtypeStruct((HIST, EMBED, BATCH), jnp.float32),
        scratch_types=[
            pltpu.VMEM((HIST, C2), jnp.int32),
            pltpu.VMEM((NSLOT, 1, C2), jnp.int32),
            pltpu.VMEM((NSLOT, C2, 2 * EMBED), jnp.float32),
            pltpu.VMEM((NSLOT, EMBED, C2), jnp.float32),
            pltpu.SemaphoreType.DMA((NSLOT,)),
            pltpu.SemaphoreType.DMA((NSLOT,)),
            pltpu.SemaphoreType.DMA,
        ],
        compiler_params=_params,
    )
    def gather_kernel(idxT, table, out, idx_v, pidx_v, rows_v, t_v, sem_g,
                      sem_w, sem_i):
        wid = lax.axis_index("s") * NC + lax.axis_index("c")
        b0 = wid * C2
        iota = lax.iota(jnp.int32, L)
        perms = [(iota + k) & (L - 1) for k in range(L)]

        pltpu.async_copy(idxT.at[pl.ds(0, HIST), pl.ds(b0, C2)], idx_v,
                         sem_i).wait()

        def start_gather(h, slot):
            for cg in range(C2 // L):
                idx = idx_v[h, pl.ds(cg * L, L)]
                pidx_v[slot, 0, pl.ds(cg * L, L)] = idx >> 1
            pltpu.async_copy(table.at[pidx_v.at[slot, 0]], rows_v.at[slot],
                             sem_g.at[slot])

        def transpose_unit(h, gslot, tslot):
            def cg_body(cg, carry):
                c_vec = cg * L + iota
                idx = idx_v[h, pl.ds(cg * L, L)]
                colbase = (idx & 1) * EMBED
                for eg in range(EMBED // L):
                    for k0 in range(0, L, 8):
                        rows = [eg * L + perms[k0 + i] for i in range(8)]
                        vals = [plsc.load_gather(rows_v.at[gslot],
                                                 [c_vec, colbase + rv])
                                for rv in rows]
                        for rv, v in zip(rows, vals):
                            plsc.store_scatter(t_v.at[tslot], [rv, c_vec],
                                               v)
                return carry
            lax.fori_loop(0, C2 // L, cg_body, 0)

        for s in range(NSLOT):
            start_gather(s, s)

        def outer(g, carry):
            for i in range(NSLOT):
                h = g * NSLOT + i

                @pl.when(h < HIST)
                def _():
                    pltpu.make_async_copy(table.at[pidx_v.at[i, 0]],
                                          rows_v.at[i],
                                          sem_g.at[i]).wait()

                    @pl.when(g > 0)
                    def _():
                        pltpu.make_async_copy(
                            t_v.at[i],
                            out.at[h, pl.ds(0, EMBED), pl.ds(b0, C2)],
                            sem_w.at[i]).wait()

                    transpose_unit(h, i, i)
                    pltpu.async_copy(
                        t_v.at[i],
                        out.at[h, pl.ds(0, EMBED), pl.ds(b0, C2)],
                        sem_w.at[i])

                    @pl.when(h + NSLOT < HIST)
                    def _():
                        start_gather(h + NSLOT, i)
            return carry

        lax.fori_loop(0, (HIST + NSLOT - 1) // NSLOT, outer, 0)

        for i in range(NSLOT):
            h_last = ((HIST - 1 - i) // NSLOT) * NSLOT + i

            @pl.when(h_last < HIST)
            def _():
                pltpu.make_async_copy(
                    t_v.at[i],
                    out.at[h_last, pl.ds(0, EMBED), pl.ds(b0, C2)],
                    sem_w.at[i]).wait()

    return gather_kernel


def kernel(input, weight):
    wt = weight.T
    wtail = jnp.pad(wt[:, NBLK * VBLK:], ((0, 0), (0, 2 * EMBED - VTAIL)))
    table = _make_transpose_kernel()(wt, wtail)
    idxT = input.T
    out = _make_gather_kernel()(idxT, table)
    return out.transpose(2, 0, 1)
